# baseline (speedup 1.0000x reference)
import functools

import jax
import jax.numpy as jnp
from jax import lax
from jax.experimental import pallas as pl
from jax.experimental.pallas import tpu as pltpu
from jax.experimental.pallas import tpu_sc as plsc


def _sc_body(y_hbm, c_hbm, t_hbm, out_hbm, y_v, c_v, t_v, out_v, sem):
    @pl.when(lax.axis_index("s") == 0)
    def _():
        d1 = pltpu.async_copy(y_hbm, y_v.at[pl.ds(0, 10)], sem)
        d2 = pltpu.async_copy(c_hbm, c_v.at[pl.ds(0, 22)], sem)
        d3 = pltpu.async_copy(t_hbm, t_v.at[pl.ds(0, 1)], sem)
        d1.wait()
        d2.wait()
        d3.wait()
        yv = y_v[pl.ds(0, 16)]
        cv = c_v[pl.ds(0, 16)]
        t = t_v[pl.ds(0, 16)][0]
        w0 = yv[9] * cv[2]
        w1 = (yv[6] + yv[8]) * cv[1]
        w2 = (yv[3] + yv[5]) * cv[0]
        w3_nodiv = cv[3] + jnp.where(t <= cv[7], cv[4], 0.0)
        num = cv[5] * t
        den = cv[8]
        lane = lax.iota(jnp.int32, 16)
        lane3 = lane == 3
        base = jnp.where(
            lane == 0,
            w0,
            jnp.where(lane == 1, w1, jnp.where(lane == 2, w2, w3_nodiv)),
        )
        res = base + jnp.where(lane3, num, 0.0) / jnp.where(lane3, den, 1.0)
        out_v[...] = res
        pltpu.sync_copy(out_v.at[pl.ds(0, 4)], out_hbm)


_sc_call = functools.partial(
    pl.kernel,
    mesh=plsc.VectorSubcoreMesh(
        core_axis_name="c", subcore_axis_name="s", num_cores=1, num_subcores=1
    ),
    out_type=jax.ShapeDtypeStruct((4,), jnp.float32),
    compiler_params=pltpu.CompilerParams(skip_device_barrier=True),
    scratch_types=[
        pltpu.VMEM((16,), jnp.float32),
        pltpu.VMEM((24,), jnp.float32),
        pltpu.VMEM((16,), jnp.float32),
        pltpu.VMEM((16,), jnp.float32),
        pltpu.SemaphoreType.DMA,
    ],
)(_sc_body)


@jax.jit
def kernel(y, w, c, t):
    return _sc_call(y, c, t.reshape(1))
